# baseline (speedup 1.0000x reference)
import jax
import jax.numpy as jnp
jax.config.update('jax_default_matmul_precision', 'highest')
from jax.experimental import pallas as pl

B = 2048
NUM_CLASSES = 100
PRE_OUT = 512
N_PART = 8
PART_LAYER = 512
NUM_DOM = 4
HIDDEN = PART_LAYER // N_PART
TAU = 0.1
EPS = 1e-5

BB = 128
GRID = B // BB


def _conv_kernel(xcol_ref, w_ref, b_ref, y_ref, st_ref):
    i = pl.program_id(0)
    y = jax.lax.dot_general(
        w_ref[...], xcol_ref[...],
        dimension_numbers=(((1,), (0,)), ((), ())),
        preferred_element_type=jnp.float32)
    y = y + b_ref[...].reshape(-1, 1)
    y_ref[...] = y
    s = y.sum(axis=1)
    sq = (y * y).sum(axis=1)
    st = jnp.stack([s, sq], axis=0)

    @pl.when(i == 0)
    def _():
        st_ref[...] = jnp.zeros_like(st_ref)

    st_ref[...] += st


def _pool_kernel(cnt):
    def body(a_ref, b_ref, c_ref, d_ref, st_ref, g_ref, bb_ref, out_ref):
        st = st_ref[...]
        m = st[0] / cnt
        v = st[1] / cnt - m * m
        scale = (g_ref[...] / jnp.sqrt(v + EPS))[:, None]
        shift = (bb_ref[...] - m * g_ref[...] / jnp.sqrt(v + EPS))[:, None]

        def nr(x):
            return jnp.maximum(x * scale + shift, 0.0)

        out_ref[...] = jnp.maximum(
            jnp.maximum(nr(a_ref[...]), nr(b_ref[...])),
            jnp.maximum(nr(c_ref[...]), nr(d_ref[...])))
    return body


def _k4(y3_ref, st3_ref, g3_ref, bb3_ref, prew_ref, preb_ref, lng_ref,
        lnb_ref, discw_ref, discb_ref, pw1_ref, pb1_ref,
        dpre_ref, hh_ref, dst_ref):
    i = pl.program_id(0)
    st = st3_ref[...]
    cnt = float(B * 8 * 8)
    m = st[0] / cnt
    v = st[1] / cnt - m * m
    scale = (g3_ref[...] / jnp.sqrt(v + EPS))[None, :, None]
    shift = (bb3_ref[...] - m * g3_ref[...] / jnp.sqrt(v + EPS))[None, :, None]
    h = jnp.maximum(y3_ref[...] * scale + shift, 0.0)
    flat = h.reshape(BB, 64 * 64)
    f = jnp.dot(flat, prew_ref[...], preferred_element_type=jnp.float32)
    f = f + preb_ref[...].reshape(1, PRE_OUT)
    m2 = f.mean(axis=1, keepdims=True)
    v2 = ((f - m2) ** 2).mean(axis=1, keepdims=True)
    f = (f - m2) / jnp.sqrt(v2 + EPS) * lng_ref[...].reshape(1, -1) \
        + lnb_ref[...].reshape(1, -1)
    f = jnp.maximum(f, 0.0)
    dpre = jnp.dot(f, discw_ref[...], preferred_element_type=jnp.float32) \
        + discb_ref[...].reshape(1, -1)
    dpre_ref[...] = dpre
    hh = jnp.dot(f, pw1_ref[...], preferred_element_type=jnp.float32) \
        + pb1_ref[...].reshape(1, -1)
    hh_ref[...] = hh
    s = dpre.sum(axis=0)
    sq = (dpre * dpre).sum(axis=0)
    st2 = jnp.stack([s, sq], axis=0)

    @pl.when(i == 0)
    def _():
        dst_ref[...] = jnp.zeros_like(dst_ref)

    dst_ref[...] += st2


def _k5(dpre_ref, dst_ref, dg_ref, db_ref, u_ref, dfcw_ref, dfcb_ref,
        sww_ref, swb_ref, hh_ref,
        dom_ref, idx_ref, probs_ref, esum_ref, esq_ref, ecnt_ref):
    i = pl.program_id(0)
    st = dst_ref[...]
    m = st[0] / float(B)
    v = st[1] / float(B) - m * m
    d = (dpre_ref[...] - m[None, :]) / jnp.sqrt(v + EPS)[None, :]
    d = jnp.maximum(d * dg_ref[...].reshape(1, -1) + db_ref[...].reshape(1, -1), 0.0)
    dom_ref[...] = jnp.dot(d, dfcw_ref[...], preferred_element_type=jnp.float32) \
        + dfcb_ref[...].reshape(1, -1)
    sw = jnp.dot(d, sww_ref[...], preferred_element_type=jnp.float32) \
        + swb_ref[...].reshape(1, -1)
    g = -jnp.log(-jnp.log(u_ref[...]))
    z = (sw + g) / TAU
    z = z - z.max(axis=1, keepdims=True)
    ez = jnp.exp(z)
    y = ez / ez.sum(axis=1, keepdims=True)
    mx = y.max(axis=1, keepdims=True)
    ids = jax.lax.broadcasted_iota(jnp.int32, (BB, N_PART), 1)
    cand = jnp.where(y == mx, ids, N_PART)
    idx = cand.min(axis=1)
    idx_ref[...] = idx[:, None]
    y_hard = (ids == idx[:, None]).astype(jnp.float32)
    probs_ref[...] = y_hard + y - y
    hh = hh_ref[...].reshape(BB, N_PART, HIDDEN)
    mask = y_hard
    esum = (hh * mask[:, :, None]).sum(axis=0)
    esq = (hh * hh * mask[:, :, None]).sum(axis=0)
    ecnt = mask.sum(axis=0)[None, :]

    @pl.when(i == 0)
    def _():
        esum_ref[...] = jnp.zeros_like(esum_ref)
        esq_ref[...] = jnp.zeros_like(esq_ref)
        ecnt_ref[...] = jnp.zeros_like(ecnt_ref)

    esum_ref[...] += esum
    esq_ref[...] += esq
    ecnt_ref[...] += ecnt


def _k6(hh_ref, idx_ref, esum_ref, esq_ref, ecnt_ref, pg_ref, pb_ref,
        pw2_ref, pb2_ref, out_ref):
    cnt = jnp.maximum(ecnt_ref[...][0], 1.0)
    m = esum_ref[...] / cnt[:, None]
    v = esq_ref[...] / cnt[:, None] - m * m
    scale = pg_ref[...] / jnp.sqrt(v + EPS)
    shift = pb_ref[...] - m * scale
    hh = hh_ref[...].reshape(BB, N_PART, HIDDEN)
    hn = jnp.maximum(hh * scale[None] + shift[None], 0.0)
    idx = idx_ref[...]
    acc = jnp.zeros((BB, NUM_CLASSES), jnp.float32)
    for e in range(N_PART):
        oo = jnp.dot(hn[:, e, :], pw2_ref[...][e],
                     preferred_element_type=jnp.float32) + pb2_ref[...][e][None, :]
        mask = (idx == e).astype(jnp.float32)
        acc = acc + mask * oo
    out_ref[...] = acc


def _full(shape):
    return pl.BlockSpec(shape, lambda i: tuple(0 for _ in shape))


def _blk(shape):
    return pl.BlockSpec(shape, lambda i: (i,) + tuple(0 for _ in shape[1:]))


def _lanes(shape):
    return pl.BlockSpec(shape, lambda i: (0, i))


def _im2col(h, c, hh, ww):
    cols = [h[:, :, dy:dy + hh - 2, dx:dx + ww - 2]
            for dy in range(3) for dx in range(3)]
    xcol = jnp.stack(cols, axis=0)
    return xcol.reshape(9 * c, B * (hh - 2) * (ww - 2))


def _conv_call(xcol, wc, bias, cout, ng):
    s = xcol.shape[1]
    return pl.pallas_call(
        _conv_kernel,
        grid=(ng,),
        in_specs=[_lanes((xcol.shape[0], s // ng)), _full(wc.shape),
                  _full(bias.shape)],
        out_specs=[_lanes((cout, s // ng)), _full((2, cout))],
        out_shape=[jax.ShapeDtypeStruct((cout, s), jnp.float32),
                   jax.ShapeDtypeStruct((2, cout), jnp.float32)],
    )(xcol, wc, bias)


def _pool_call(y, st, g, b, c, hh, ww, cnt, ng):
    yr = y.reshape(c, B, hh, ww)
    parts = [yr[:, :, 0::2, 0::2], yr[:, :, 0::2, 1::2],
             yr[:, :, 1::2, 0::2], yr[:, :, 1::2, 1::2]]
    parts = [p.reshape(c, B * (hh // 2) * (ww // 2)) for p in parts]
    s = parts[0].shape[1]
    return pl.pallas_call(
        _pool_kernel(cnt),
        grid=(ng,),
        in_specs=[_lanes((c, s // ng))] * 4 + [_full((2, c)), _full((c,)),
                                               _full((c,))],
        out_specs=[_lanes((c, s // ng))],
        out_shape=[jax.ShapeDtypeStruct((c, s), jnp.float32)],
    )(*parts, st, g, b)[0]


@jax.jit
def _run(input_data, params, u):
    p = params
    f32 = jnp.float32

    w1c = p['conv1_w'].transpose(0, 2, 3, 1).reshape(16, 27)
    w2c = p['conv2_w'].transpose(0, 2, 3, 1).reshape(32, 144)
    w3c = p['conv3_w'].transpose(0, 2, 3, 1).reshape(64, 288)

    xp = jnp.pad(input_data.transpose(1, 0, 2, 3),
                 ((0, 0), (0, 0), (1, 1), (1, 1)))
    xcol1 = _im2col(xp, 3, 34, 34)
    y1, st1 = _conv_call(xcol1, w1c, p['conv1_b'], 16, 32)

    h1 = _pool_call(y1, st1, p['bn1_g'], p['bn1_b'], 16, 32, 32,
                    float(B * 1024), 16)
    h1p = jnp.pad(h1.reshape(16, B, 16, 16),
                  ((0, 0), (0, 0), (1, 1), (1, 1)))
    xcol2 = _im2col(h1p, 16, 18, 18)
    y2, st2 = _conv_call(xcol2, w2c, p['conv2_b'], 32, 16)

    h2 = _pool_call(y2, st2, p['bn2_g'], p['bn2_b'], 32, 16, 16,
                    float(B * 256), 8)
    h2p = jnp.pad(h2.reshape(32, B, 8, 8),
                  ((0, 0), (0, 0), (1, 1), (1, 1)))
    xcol3 = _im2col(h2p, 32, 10, 10)
    y3, st3 = _conv_call(xcol3, w3c, p['conv3_b'], 64, 8)

    y3b = y3.reshape(64, B, 64).transpose(1, 0, 2)

    dpre, hh, dst = pl.pallas_call(
        _k4,
        grid=(GRID,),
        in_specs=[_blk((BB, 64, 64)), _full((2, 64)), _full((64,)),
                  _full((64,)), _full((4096, 512)), _full((512,)),
                  _full((512,)), _full((512,)), _full((512, 512)),
                  _full((512,)), _full((512, 512)), _full((512,))],
        out_specs=[_blk((BB, 512)), _blk((BB, 512)), _full((2, 512))],
        out_shape=[jax.ShapeDtypeStruct((B, 512), f32),
                   jax.ShapeDtypeStruct((B, 512), f32),
                   jax.ShapeDtypeStruct((2, 512), f32)],
    )(y3b, st3, p['bn3_g'], p['bn3_b'], p['pre_w'].T, p['pre_b'],
      p['ln_g'], p['ln_b'], p['disc_w'].T, p['disc_b'],
      p['pw1'].reshape(512, 512).T, p['pb1'].reshape(512))

    dom, idx, probs, esum, esq, ecnt = pl.pallas_call(
        _k5,
        grid=(GRID,),
        in_specs=[_blk((BB, 512)), _full((2, 512)), _full((512,)),
                  _full((512,)), _blk((BB, N_PART)), _full((512, NUM_DOM)),
                  _full((NUM_DOM,)), _full((512, N_PART)),
                  _full((N_PART,)), _blk((BB, 512))],
        out_specs=[_blk((BB, NUM_DOM)), _blk((BB, 1)), _blk((BB, N_PART)),
                   _full((N_PART, HIDDEN)), _full((N_PART, HIDDEN)),
                   _full((1, N_PART))],
        out_shape=[jax.ShapeDtypeStruct((B, NUM_DOM), f32),
                   jax.ShapeDtypeStruct((B, 1), jnp.int32),
                   jax.ShapeDtypeStruct((B, N_PART), f32),
                   jax.ShapeDtypeStruct((N_PART, HIDDEN), f32),
                   jax.ShapeDtypeStruct((N_PART, HIDDEN), f32),
                   jax.ShapeDtypeStruct((1, N_PART), f32)],
    )(dpre, dst, p['dbn_g'], p['dbn_b'], u, p['dfc_w'].T, p['dfc_b'],
      p['sw_w'].T, p['sw_b'], hh)

    out = pl.pallas_call(
        _k6,
        grid=(GRID,),
        in_specs=[_blk((BB, 512)), _blk((BB, 1)), _full((N_PART, HIDDEN)),
                  _full((N_PART, HIDDEN)), _full((1, N_PART)),
                  _full((N_PART, HIDDEN)), _full((N_PART, HIDDEN)),
                  _full((N_PART, HIDDEN, NUM_CLASSES)),
                  _full((N_PART, NUM_CLASSES))],
        out_specs=[_blk((BB, NUM_CLASSES))],
        out_shape=[jax.ShapeDtypeStruct((B, NUM_CLASSES), f32)],
    )(hh, idx, esum, esq, ecnt, p['pbn_g'], p['pbn_b'],
      p['pw2'].transpose(0, 2, 1), p['pb2'])

    return out[0], dom, idx.reshape(B), probs


def kernel(input_data, params, u):
    return _run(input_data, params, u)
